# baseline (speedup 1.0000x reference)
import functools

import jax
import jax.numpy as jnp
from jax import lax
from jax.experimental import pallas as pl
from jax.experimental.pallas import tpu as pltpu
from jax.experimental.pallas import tpu_sc as plsc

_B, _L, _EMB = 1024, 50, 128
_MEM = 65536
_K = 256
_MARGIN = 0.1
_NW = 32


def _sc_gather(table, idx, chunk_sizes):
    n = idx.shape[0]
    d = table.shape[1]
    b_per_w = n // _NW
    assert sum(chunk_sizes) == b_per_w
    mesh = plsc.VectorSubcoreMesh(core_axis_name="c", subcore_axis_name="s")
    uniq = sorted(set(chunk_sizes), reverse=True)
    buf_of = {u: i for i, u in enumerate(uniq)}
    scratch = []
    for u in uniq:
        scratch.append(pltpu.VMEM((u,), jnp.int32))
        scratch.append(pltpu.VMEM((u, d), jnp.float32))
    scratch.append(pltpu.SemaphoreType.DMA)

    def body(table_hbm, idx_hbm, out_hbm, *rest):
        sem = rest[-1]
        wid = lax.axis_index("s") * 2 + lax.axis_index("c")
        base = wid * b_per_w
        off = 0
        for cs in chunk_sizes:
            iv = rest[2 * buf_of[cs]]
            rv = rest[2 * buf_of[cs] + 1]
            pltpu.sync_copy(idx_hbm.at[pl.ds(base + off, cs)], iv)
            pltpu.async_copy(table_hbm.at[iv], rv, sem).wait()
            pltpu.sync_copy(rv, out_hbm.at[pl.ds(base + off, cs)])
            off += cs

    k = pl.kernel(
        body,
        mesh=mesh,
        out_type=jax.ShapeDtypeStruct((n, d), jnp.float32),
        scratch_types=scratch,
    )
    return k(table, idx)


def _sc_gather_emb(table, idx):
    n = idx.shape[0]
    d = table.shape[1]
    b_per_w = n // _NW
    nch = 12
    mesh = plsc.VectorSubcoreMesh(core_axis_name="c", subcore_axis_name="s")
    scratch = [
        pltpu.VMEM((128,), jnp.int32), pltpu.VMEM((128,), jnp.int32),
        pltpu.VMEM((128, d), jnp.float32), pltpu.VMEM((128, d), jnp.float32),
        pltpu.VMEM((64,), jnp.int32), pltpu.VMEM((64, d), jnp.float32),
        pltpu.SemaphoreType.DMA, pltpu.SemaphoreType.DMA,
        pltpu.SemaphoreType.DMA, pltpu.SemaphoreType.DMA,
        pltpu.SemaphoreType.DMA, pltpu.SemaphoreType.DMA,
        pltpu.SemaphoreType.DMA,
    ]

    def body(table_hbm, idx_hbm, out_hbm, iv0, iv1, rv0, rv1, ivt, rvt,
             si0, si1, sg0, sg1, so0, so1, st):
        iv, rv = (iv0, iv1), (rv0, rv1)
        si, sg, so = (si0, si1), (sg0, sg1), (so0, so1)
        wid = lax.axis_index("s") * 2 + lax.axis_index("c")
        base = wid * b_per_w
        idx_cp = [None] * nch
        g_cp = [None] * nch
        o_cp = [None] * nch
        idx_cp[0] = pltpu.async_copy(idx_hbm.at[pl.ds(base, 128)], iv[0],
                                     si[0])
        for c in range(nch):
            p = c & 1
            idx_cp[c].wait()
            if c >= 2:
                o_cp[c - 2].wait()
            g_cp[c] = pltpu.async_copy(table_hbm.at[iv[p]], rv[p], sg[p])
            if c + 1 < nch:
                idx_cp[c + 1] = pltpu.async_copy(
                    idx_hbm.at[pl.ds(base + (c + 1) * 128, 128)], iv[1 - p],
                    si[1 - p])
            g_cp[c].wait()
            o_cp[c] = pltpu.async_copy(
                rv[p], out_hbm.at[pl.ds(base + c * 128, 128)], so[p])
        pltpu.sync_copy(idx_hbm.at[pl.ds(base + nch * 128, 64)], ivt)
        pltpu.async_copy(table_hbm.at[ivt], rvt, st).wait()
        pltpu.sync_copy(rvt, out_hbm.at[pl.ds(base + nch * 128, 64)])
        o_cp[nch - 2].wait()
        o_cp[nch - 1].wait()

    k = pl.kernel(
        body,
        mesh=mesh,
        out_type=jax.ShapeDtypeStruct((n, d), jnp.float32),
        scratch_types=scratch,
    )
    return k(table, idx)


_SB = 64
_TAPS = 12


_LP = 56


def _cnn_body(xe_ref, wcat_ref, bias_ref, fcw_ref, q_ref, xep):
    for b in range(_SB):
        xep[b * _LP:b * _LP + _L, :] = xe_ref[b * _L:(b + 1) * _L, :]
    p = lax.dot_general(xep[:], wcat_ref[:], (((1,), (0,)), ((), ())),
                        preferred_element_type=jnp.float32)
    p = p.reshape(_SB, _LP, _TAPS * 128)

    def conv(tap0, ntaps, brow):
        npos = _L - ntaps + 1
        acc = None
        for t in range(ntaps):
            sl = p[:, t:t + npos, (tap0 + t) * 128:(tap0 + t + 1) * 128]
            acc = sl if acc is None else acc + sl
        acc = acc + bias_ref[brow:brow + 1, :].reshape(1, 1, 128)
        return jnp.max(jnp.maximum(acc, 0.0), axis=1)

    f3 = conv(0, 3, 0)
    f4 = conv(3, 4, 1)
    f5 = conv(7, 5, 2)
    f = jnp.concatenate([f3, f4, f5], axis=1)
    z = lax.dot_general(f, fcw_ref[:], (((1,), (0,)), ((), ())),
                        preferred_element_type=jnp.float32) + bias_ref[3:4, :]
    nrm = jnp.sqrt(jnp.sum(z * z, axis=1, keepdims=True))
    q_ref[:] = z / (nrm + 1e-8)


_MBLK = 2048
_NMB = _MEM // _MBLK


def _mem_body(q_ref, mk_ref, vals_ref, y_ref, io_ref,
              loss_ref, acc_ref, n1_ref, corr_ref,
              mh, mn, ia, ch, cn):
    ph = pl.program_id(0)
    j = pl.program_id(1)

    @pl.when((ph == 0) & (j == 0))
    def _():
        mh[:] = jnp.full((1, _B), -3.0, jnp.float32)
        mn[:] = jnp.full((1, _B), -3.0, jnp.float32)

    mk = mk_ref[pl.ds(j * _MBLK, _MBLK), :]
    s = lax.dot_general(mk, q_ref[:], (((1,), (1,)), ((), ())),
                        preferred_element_type=jnp.float32)

    @pl.when(ph == 0)
    def _():
        hit = vals_ref[:] == y_ref[:]
        mh[:] = jnp.maximum(jnp.max(jnp.where(hit, s, -2.0), axis=0,
                                    keepdims=True), mh[:])
        mn[:] = jnp.maximum(jnp.max(jnp.where(hit, -2.0, s), axis=0,
                                    keepdims=True), mn[:])

    @pl.when((ph == 1) & (j == 0))
    def _():
        ia[:] = jnp.full((1, _B), 2 ** 30, jnp.int32)
        ch[:] = jnp.zeros((1, _B), jnp.float32)
        cn[:] = jnp.zeros((1, _B), jnp.float32)

    @pl.when(ph == 1)
    def _():
        big = jnp.int32(2 ** 30)
        ma = jnp.maximum(mh[:], mn[:])
        bidx = jnp.min(jnp.where(s == ma, io_ref[:], big), axis=0,
                       keepdims=True)
        ia[:] = jnp.minimum(ia[:], bidx + j * _MBLK)
        ch[:] = ch[:] + jnp.sum((s > mh[:]).astype(jnp.float32), axis=0,
                                keepdims=True)
        cn[:] = cn[:] + jnp.sum((s > mn[:]).astype(jnp.float32), axis=0,
                                keepdims=True)

    @pl.when((ph == 1) & (j == _NMB - 1))
    def _():
        mhv, mnv = mh[:], mn[:]
        corr = mhv > mnv
        kf = jnp.float32(_K) - 0.5
        pos = jnp.where(ch[:] < kf, mhv, 0.0)
        neg = jnp.where(cn[:] < kf, mnv, -1e9)
        lossv = jnp.maximum(neg - pos + _MARGIN, 0.0)
        loss_ref[:] = (jnp.sum(lossv) / _B).reshape(1, 1)
        acc_ref[:] = (jnp.sum(corr.astype(jnp.float32)) / _B).reshape(1, 1)
        n1_ref[:] = ia[:]
        corr_ref[:] = corr.astype(jnp.int32)


_SBLK = 4096
_NSB = _MEM // _SBLK


def _scatter_body(mk_ref, q_ref, mk1_ref, corr_ref, n1_ref, y_ref,
                  n1r_ref, corrr_ref, yr_ref, out_ref, newk, wrow_s):
    step = pl.program_id(0)

    @pl.when(step == 0)
    def _():
        qv = q_ref[:]
        upd = qv + mk1_ref[:]
        nrm = jnp.sqrt(jnp.sum(upd * upd, axis=1, keepdims=True))
        upd = upd / (nrm + 1e-8)
        corr_c = corr_ref[:] != 0
        newk[:] = jnp.where(corr_c, upd, qv)
        lane = lax.broadcasted_iota(jnp.int32, (1, _B), 1)
        sub = lax.broadcasted_iota(jnp.int32, (_B, 1), 0)
        wrow = jnp.where(corrr_ref[:] != 0, n1r_ref[:],
                         (yr_ref[:] * 6151 + lane) % _MEM)
        wcol = jnp.where(corr_c, n1_ref[:],
                         (y_ref[:] * 6151 + sub) % _MEM)
        match = wcol == wrow
        sub2 = lax.broadcasted_iota(jnp.int32, (_B, _B), 0)
        lastk = jnp.max(jnp.where(match, sub2, -1), axis=0, keepdims=True)
        wrow_s[:] = jnp.where(lastk == lane, wrow, -1)

    slot = lax.broadcasted_iota(jnp.int32, (_SBLK, 1), 0) + step * _SBLK
    match = slot == wrow_s[:]
    oh = match.astype(jnp.float32)
    sel = lax.dot_general(oh, newk[:], (((1,), (0,)), ((), ())),
                          preferred_element_type=jnp.float32)
    upd_row = jnp.any(match, axis=1, keepdims=True)
    out_ref[:] = jnp.where(upd_row, sel, mk_ref[:])


def _pad_tap(w):
    return jnp.pad(w, ((0, 0), (0, 128 - w.shape[1])))


def kernel(x, y, emb, w3, b3, w4, b4, w5, b5, fcW, fcb, mem_keys, mem_vals):
    f32 = jnp.float32

    xe = _sc_gather_emb(emb, x.reshape(-1))

    taps = [w3[t] for t in range(3)] + [w4[t] for t in range(4)] + \
           [w5[t] for t in range(5)]
    wcat = jnp.concatenate([_pad_tap(t) for t in taps], axis=1)
    pad1 = 128 - b3.shape[0]
    bias = jnp.stack([
        jnp.pad(b3, (0, pad1)), jnp.pad(b4, (0, pad1)), jnp.pad(b5, (0, pad1)),
        fcb, jnp.zeros((128,), f32), jnp.zeros((128,), f32),
        jnp.zeros((128,), f32), jnp.zeros((128,), f32)])
    kn = b3.shape[0]
    fcwp = jnp.concatenate([
        jnp.pad(fcW[0:kn], ((0, 128 - kn), (0, 0))),
        jnp.pad(fcW[kn:2 * kn], ((0, 128 - kn), (0, 0))),
        jnp.pad(fcW[2 * kn:3 * kn], ((0, 128 - kn), (0, 0)))], axis=0)

    q = pl.pallas_call(
        _cnn_body,
        grid=(_B // _SB,),
        in_specs=[
            pl.BlockSpec((_SB * _L, 128), lambda i: (i, 0)),
            pl.BlockSpec((128, _TAPS * 128), lambda i: (0, 0)),
            pl.BlockSpec((8, 128), lambda i: (0, 0)),
            pl.BlockSpec((384, 128), lambda i: (0, 0)),
        ],
        out_specs=pl.BlockSpec((_SB, 128), lambda i: (i, 0)),
        out_shape=jax.ShapeDtypeStruct((_B, 128), f32),
        scratch_shapes=[pltpu.VMEM((_SB * _LP, 128), f32)],
    )(xe, wcat, bias, fcwp)

    vals2 = mem_vals.reshape(_MEM, 1)
    y2 = y.reshape(_B, 1)
    yr = y.reshape(1, _B)
    io_col = jnp.arange(_MBLK, dtype=jnp.int32).reshape(_MBLK, 1)
    loss2, acc2, n1r, corrr = pl.pallas_call(
        _mem_body,
        grid=(2, _NMB),
        in_specs=[
            pl.BlockSpec((_B, 128), lambda p, j: (0, 0)),
            pl.BlockSpec((_MEM, 128), lambda p, j: (0, 0)),
            pl.BlockSpec((_MBLK, 1), lambda p, j: (j, 0)),
            pl.BlockSpec((1, _B), lambda p, j: (0, 0)),
            pl.BlockSpec((_MBLK, 1), lambda p, j: (0, 0)),
        ],
        out_specs=[
            pl.BlockSpec((1, 1), lambda p, j: (0, 0)),
            pl.BlockSpec((1, 1), lambda p, j: (0, 0)),
            pl.BlockSpec((1, _B), lambda p, j: (0, 0)),
            pl.BlockSpec((1, _B), lambda p, j: (0, 0)),
        ],
        out_shape=[
            jax.ShapeDtypeStruct((1, 1), f32),
            jax.ShapeDtypeStruct((1, 1), f32),
            jax.ShapeDtypeStruct((1, _B), jnp.int32),
            jax.ShapeDtypeStruct((1, _B), jnp.int32),
        ],
        scratch_shapes=[
            pltpu.VMEM((1, _B), f32), pltpu.VMEM((1, _B), f32),
            pltpu.VMEM((1, _B), jnp.int32),
            pltpu.VMEM((1, _B), f32), pltpu.VMEM((1, _B), f32),
        ],
    )(q, mem_keys, vals2, yr, io_col)

    mk1 = _sc_gather(mem_keys, n1r.reshape(-1), [32])

    new_mem_keys = pl.pallas_call(
        _scatter_body,
        grid=(_NSB,),
        in_specs=[
            pl.BlockSpec((_SBLK, 128), lambda jj: (jj, 0)),
            pl.BlockSpec((_B, 128), lambda jj: (0, 0)),
            pl.BlockSpec((_B, 128), lambda jj: (0, 0)),
            pl.BlockSpec((_B, 1), lambda jj: (0, 0)),
            pl.BlockSpec((_B, 1), lambda jj: (0, 0)),
            pl.BlockSpec((_B, 1), lambda jj: (0, 0)),
            pl.BlockSpec((1, _B), lambda jj: (0, 0)),
            pl.BlockSpec((1, _B), lambda jj: (0, 0)),
            pl.BlockSpec((1, _B), lambda jj: (0, 0)),
        ],
        out_specs=pl.BlockSpec((_SBLK, 128), lambda jj: (jj, 0)),
        out_shape=jax.ShapeDtypeStruct((_MEM, 128), f32),
        scratch_shapes=[
            pltpu.VMEM((_B, 128), f32), pltpu.VMEM((1, _B), jnp.int32),
        ],
    )(mem_keys, q, mk1, corrr.reshape(_B, 1), n1r.reshape(_B, 1), y2,
      n1r, corrr, yr)

    return (loss2[0, 0], acc2[0, 0], new_mem_keys)


# speedup vs baseline: 1.0469x; 1.0469x over previous
import functools

import jax
import jax.numpy as jnp
from jax import lax
from jax.experimental import pallas as pl
from jax.experimental.pallas import tpu as pltpu
from jax.experimental.pallas import tpu_sc as plsc

_B, _L, _EMB = 1024, 50, 128
_MEM = 65536
_K = 256
_MARGIN = 0.1
_NW = 32


def _sc_gather(table, idx, chunk_sizes):
    n = idx.shape[0]
    d = table.shape[1]
    b_per_w = n // _NW
    assert sum(chunk_sizes) == b_per_w
    mesh = plsc.VectorSubcoreMesh(core_axis_name="c", subcore_axis_name="s")
    uniq = sorted(set(chunk_sizes), reverse=True)
    buf_of = {u: i for i, u in enumerate(uniq)}
    scratch = []
    for u in uniq:
        scratch.append(pltpu.VMEM((u,), jnp.int32))
        scratch.append(pltpu.VMEM((u, d), jnp.float32))
    scratch.append(pltpu.SemaphoreType.DMA)

    def body(table_hbm, idx_hbm, out_hbm, *rest):
        sem = rest[-1]
        wid = lax.axis_index("s") * 2 + lax.axis_index("c")
        base = wid * b_per_w
        off = 0
        for cs in chunk_sizes:
            iv = rest[2 * buf_of[cs]]
            rv = rest[2 * buf_of[cs] + 1]
            pltpu.sync_copy(idx_hbm.at[pl.ds(base + off, cs)], iv)
            pltpu.async_copy(table_hbm.at[iv], rv, sem).wait()
            pltpu.sync_copy(rv, out_hbm.at[pl.ds(base + off, cs)])
            off += cs

    k = pl.kernel(
        body,
        mesh=mesh,
        out_type=jax.ShapeDtypeStruct((n, d), jnp.float32),
        scratch_types=scratch,
    )
    return k(table, idx)


def _sc_gather_emb(table, idx):
    n = idx.shape[0]
    d = table.shape[1]
    b_per_w = n // _NW
    nch = 12
    mesh = plsc.VectorSubcoreMesh(core_axis_name="c", subcore_axis_name="s")
    scratch = [
        pltpu.VMEM((128,), jnp.int32), pltpu.VMEM((128,), jnp.int32),
        pltpu.VMEM((128, d), jnp.float32), pltpu.VMEM((128, d), jnp.float32),
        pltpu.VMEM((64,), jnp.int32), pltpu.VMEM((64, d), jnp.float32),
        pltpu.SemaphoreType.DMA, pltpu.SemaphoreType.DMA,
        pltpu.SemaphoreType.DMA, pltpu.SemaphoreType.DMA,
        pltpu.SemaphoreType.DMA, pltpu.SemaphoreType.DMA,
        pltpu.SemaphoreType.DMA,
    ]

    def body(table_hbm, idx_hbm, out_hbm, iv0, iv1, rv0, rv1, ivt, rvt,
             si0, si1, sg0, sg1, so0, so1, st):
        iv, rv = (iv0, iv1), (rv0, rv1)
        si, sg, so = (si0, si1), (sg0, sg1), (so0, so1)
        wid = lax.axis_index("s") * 2 + lax.axis_index("c")
        base = wid * b_per_w
        idx_cp = [None] * nch
        g_cp = [None] * nch
        o_cp = [None] * nch
        idx_cp[0] = pltpu.async_copy(idx_hbm.at[pl.ds(base, 128)], iv[0],
                                     si[0])
        for c in range(nch):
            p = c & 1
            idx_cp[c].wait()
            if c >= 2:
                o_cp[c - 2].wait()
            g_cp[c] = pltpu.async_copy(table_hbm.at[iv[p]], rv[p], sg[p])
            if c + 1 < nch:
                idx_cp[c + 1] = pltpu.async_copy(
                    idx_hbm.at[pl.ds(base + (c + 1) * 128, 128)], iv[1 - p],
                    si[1 - p])
            g_cp[c].wait()
            o_cp[c] = pltpu.async_copy(
                rv[p], out_hbm.at[pl.ds(base + c * 128, 128)], so[p])
        pltpu.sync_copy(idx_hbm.at[pl.ds(base + nch * 128, 64)], ivt)
        pltpu.async_copy(table_hbm.at[ivt], rvt, st).wait()
        pltpu.sync_copy(rvt, out_hbm.at[pl.ds(base + nch * 128, 64)])
        o_cp[nch - 2].wait()
        o_cp[nch - 1].wait()

    k = pl.kernel(
        body,
        mesh=mesh,
        out_type=jax.ShapeDtypeStruct((n, d), jnp.float32),
        scratch_types=scratch,
    )
    return k(table, idx)


_SB = 64
_TAPS = 12


_LP = 56


def _cnn_body(xe_ref, wcat_ref, bias_ref, fcw_ref, q_ref, xep):
    for b in range(_SB):
        xep[b * _LP:b * _LP + _L, :] = xe_ref[b * _L:(b + 1) * _L, :]
    p = lax.dot_general(xep[:], wcat_ref[:], (((1,), (0,)), ((), ())),
                        preferred_element_type=jnp.float32)
    p = p.reshape(_SB, _LP, _TAPS * 128)

    def conv(tap0, ntaps, brow):
        npos = _L - ntaps + 1
        acc = None
        for t in range(ntaps):
            sl = p[:, t:t + npos, (tap0 + t) * 128:(tap0 + t + 1) * 128]
            acc = sl if acc is None else acc + sl
        acc = acc + bias_ref[brow:brow + 1, :].reshape(1, 1, 128)
        return jnp.max(jnp.maximum(acc, 0.0), axis=1)

    f3 = conv(0, 3, 0)
    f4 = conv(3, 4, 1)
    f5 = conv(7, 5, 2)
    f = jnp.concatenate([f3, f4, f5], axis=1)
    z = lax.dot_general(f, fcw_ref[:], (((1,), (0,)), ((), ())),
                        preferred_element_type=jnp.float32) + bias_ref[3:4, :]
    nrm = jnp.sqrt(jnp.sum(z * z, axis=1, keepdims=True))
    q_ref[:] = z / (nrm + 1e-8)


_MBLK = 4096
_NMB = _MEM // _MBLK


def _mem_body(q_ref, mk_ref, vals_ref, y_ref, io_ref,
              loss_ref, acc_ref, n1_ref, corr_ref,
              mh, mn, ia, ch, cn):
    ph = pl.program_id(0)
    j = pl.program_id(1)

    @pl.when((ph == 0) & (j == 0))
    def _():
        mh[:] = jnp.full((1, _B), -3.0, jnp.float32)
        mn[:] = jnp.full((1, _B), -3.0, jnp.float32)

    s = lax.dot_general(mk_ref[:], q_ref[:], (((1,), (1,)), ((), ())),
                        preferred_element_type=jnp.float32)

    @pl.when(ph == 0)
    def _():
        hit = vals_ref[:] == y_ref[:]
        mh[:] = jnp.maximum(jnp.max(jnp.where(hit, s, -2.0), axis=0,
                                    keepdims=True), mh[:])
        mn[:] = jnp.maximum(jnp.max(jnp.where(hit, -2.0, s), axis=0,
                                    keepdims=True), mn[:])

    @pl.when((ph == 1) & (j == 0))
    def _():
        ia[:] = jnp.full((1, _B), 2 ** 30, jnp.int32)
        ch[:] = jnp.zeros((1, _B), jnp.float32)
        cn[:] = jnp.zeros((1, _B), jnp.float32)

    @pl.when(ph == 1)
    def _():
        big = jnp.int32(2 ** 30)
        ma = jnp.maximum(mh[:], mn[:])
        bidx = jnp.min(jnp.where(s == ma, io_ref[:], big), axis=0,
                       keepdims=True)
        ia[:] = jnp.minimum(ia[:], bidx + j * _MBLK)
        ch[:] = ch[:] + jnp.sum((s > mh[:]).astype(jnp.float32), axis=0,
                                keepdims=True)
        cn[:] = cn[:] + jnp.sum((s > mn[:]).astype(jnp.float32), axis=0,
                                keepdims=True)

    @pl.when((ph == 1) & (j == _NMB - 1))
    def _():
        mhv, mnv = mh[:], mn[:]
        corr = mhv > mnv
        kf = jnp.float32(_K) - 0.5
        pos = jnp.where(ch[:] < kf, mhv, 0.0)
        neg = jnp.where(cn[:] < kf, mnv, -1e9)
        lossv = jnp.maximum(neg - pos + _MARGIN, 0.0)
        loss_ref[:] = (jnp.sum(lossv) / _B).reshape(1, 1)
        acc_ref[:] = (jnp.sum(corr.astype(jnp.float32)) / _B).reshape(1, 1)
        n1_ref[:] = ia[:]
        corr_ref[:] = corr.astype(jnp.int32)


_SBLK = 4096
_NSB = _MEM // _SBLK


def _scatter_body(mk_ref, q_ref, mk1_ref, corr_ref, n1_ref, y_ref,
                  n1r_ref, corrr_ref, yr_ref, out_ref, newk, wrow_s):
    step = pl.program_id(0)

    @pl.when(step == 0)
    def _():
        qv = q_ref[:]
        upd = qv + mk1_ref[:]
        nrm = jnp.sqrt(jnp.sum(upd * upd, axis=1, keepdims=True))
        upd = upd / (nrm + 1e-8)
        corr_c = corr_ref[:] != 0
        newk[:] = jnp.where(corr_c, upd, qv)
        lane = lax.broadcasted_iota(jnp.int32, (1, _B), 1)
        sub = lax.broadcasted_iota(jnp.int32, (_B, 1), 0)
        wrow = jnp.where(corrr_ref[:] != 0, n1r_ref[:],
                         (yr_ref[:] * 6151 + lane) % _MEM)
        wcol = jnp.where(corr_c, n1_ref[:],
                         (y_ref[:] * 6151 + sub) % _MEM)
        match = wcol == wrow
        sub2 = lax.broadcasted_iota(jnp.int32, (_B, _B), 0)
        lastk = jnp.max(jnp.where(match, sub2, -1), axis=0, keepdims=True)
        wrow_s[:] = jnp.where(lastk == lane, wrow, -1)

    slot = lax.broadcasted_iota(jnp.int32, (_SBLK, 1), 0) + step * _SBLK
    match = slot == wrow_s[:]
    oh = match.astype(jnp.float32)
    sel = lax.dot_general(oh, newk[:], (((1,), (0,)), ((), ())),
                          preferred_element_type=jnp.float32)
    upd_row = jnp.any(match, axis=1, keepdims=True)
    out_ref[:] = jnp.where(upd_row, sel, mk_ref[:])


def _pad_tap(w):
    return jnp.pad(w, ((0, 0), (0, 128 - w.shape[1])))


def kernel(x, y, emb, w3, b3, w4, b4, w5, b5, fcW, fcb, mem_keys, mem_vals):
    f32 = jnp.float32

    xe = _sc_gather_emb(emb, x.reshape(-1))

    taps = [w3[t] for t in range(3)] + [w4[t] for t in range(4)] + \
           [w5[t] for t in range(5)]
    wcat = jnp.concatenate([_pad_tap(t) for t in taps], axis=1)
    pad1 = 128 - b3.shape[0]
    bias = jnp.stack([
        jnp.pad(b3, (0, pad1)), jnp.pad(b4, (0, pad1)), jnp.pad(b5, (0, pad1)),
        fcb, jnp.zeros((128,), f32), jnp.zeros((128,), f32),
        jnp.zeros((128,), f32), jnp.zeros((128,), f32)])
    kn = b3.shape[0]
    fcwp = jnp.concatenate([
        jnp.pad(fcW[0:kn], ((0, 128 - kn), (0, 0))),
        jnp.pad(fcW[kn:2 * kn], ((0, 128 - kn), (0, 0))),
        jnp.pad(fcW[2 * kn:3 * kn], ((0, 128 - kn), (0, 0)))], axis=0)

    q = pl.pallas_call(
        _cnn_body,
        grid=(_B // _SB,),
        in_specs=[
            pl.BlockSpec((_SB * _L, 128), lambda i: (i, 0)),
            pl.BlockSpec((128, _TAPS * 128), lambda i: (0, 0)),
            pl.BlockSpec((8, 128), lambda i: (0, 0)),
            pl.BlockSpec((384, 128), lambda i: (0, 0)),
        ],
        out_specs=pl.BlockSpec((_SB, 128), lambda i: (i, 0)),
        out_shape=jax.ShapeDtypeStruct((_B, 128), f32),
        scratch_shapes=[pltpu.VMEM((_SB * _LP, 128), f32)],
    )(xe, wcat, bias, fcwp)

    vals2 = mem_vals.reshape(_MEM, 1)
    y2 = y.reshape(_B, 1)
    yr = y.reshape(1, _B)
    io_col = jnp.arange(_MBLK, dtype=jnp.int32).reshape(_MBLK, 1)
    loss2, acc2, n1r, corrr = pl.pallas_call(
        _mem_body,
        grid=(2, _NMB),
        in_specs=[
            pl.BlockSpec((_B, 128), lambda p, j: (0, 0)),
            pl.BlockSpec((_MBLK, 128), lambda p, j: (j, 0)),
            pl.BlockSpec((_MBLK, 1), lambda p, j: (j, 0)),
            pl.BlockSpec((1, _B), lambda p, j: (0, 0)),
            pl.BlockSpec((_MBLK, 1), lambda p, j: (0, 0)),
        ],
        out_specs=[
            pl.BlockSpec((1, 1), lambda p, j: (0, 0)),
            pl.BlockSpec((1, 1), lambda p, j: (0, 0)),
            pl.BlockSpec((1, _B), lambda p, j: (0, 0)),
            pl.BlockSpec((1, _B), lambda p, j: (0, 0)),
        ],
        out_shape=[
            jax.ShapeDtypeStruct((1, 1), f32),
            jax.ShapeDtypeStruct((1, 1), f32),
            jax.ShapeDtypeStruct((1, _B), jnp.int32),
            jax.ShapeDtypeStruct((1, _B), jnp.int32),
        ],
        scratch_shapes=[
            pltpu.VMEM((1, _B), f32), pltpu.VMEM((1, _B), f32),
            pltpu.VMEM((1, _B), jnp.int32),
            pltpu.VMEM((1, _B), f32), pltpu.VMEM((1, _B), f32),
        ],
    )(q, mem_keys, vals2, yr, io_col)

    mk1 = _sc_gather(mem_keys, n1r.reshape(-1), [32])

    new_mem_keys = pl.pallas_call(
        _scatter_body,
        grid=(_NSB,),
        in_specs=[
            pl.BlockSpec((_SBLK, 128), lambda jj: (jj, 0)),
            pl.BlockSpec((_B, 128), lambda jj: (0, 0)),
            pl.BlockSpec((_B, 128), lambda jj: (0, 0)),
            pl.BlockSpec((_B, 1), lambda jj: (0, 0)),
            pl.BlockSpec((_B, 1), lambda jj: (0, 0)),
            pl.BlockSpec((_B, 1), lambda jj: (0, 0)),
            pl.BlockSpec((1, _B), lambda jj: (0, 0)),
            pl.BlockSpec((1, _B), lambda jj: (0, 0)),
            pl.BlockSpec((1, _B), lambda jj: (0, 0)),
        ],
        out_specs=pl.BlockSpec((_SBLK, 128), lambda jj: (jj, 0)),
        out_shape=jax.ShapeDtypeStruct((_MEM, 128), f32),
        scratch_shapes=[
            pltpu.VMEM((_B, 128), f32), pltpu.VMEM((1, _B), jnp.int32),
        ],
    )(mem_keys, q, mk1, corrr.reshape(_B, 1), n1r.reshape(_B, 1), y2,
      n1r, corrr, yr)

    return (loss2[0, 0], acc2[0, 0], new_mem_keys)
